# baseline (speedup 1.0000x reference)
import dataclasses

import jax
import jax.numpy as jnp
from jax import lax
from jax.experimental import pallas as pl
from jax.experimental.pallas import tpu as pltpu
from jax.experimental.pallas import tpu_sc as plsc

N = 100000
NC, NS, L = 2, 16, 16
NW = NC * NS
VECS = N // L
MAIN_VECS = VECS // NW
CHUNK = MAIN_VECS * L
HALF_VECS_A = MAIN_VECS // 2 + 1
HALF_A = HALF_VECS_A * L
HALF_VECS_B = MAIN_VECS - HALF_VECS_A
HALF_B = HALF_VECS_B * L
TAIL_VECS = VECS - MAIN_VECS * NW
TAIL_BASE = NW * CHUNK
TBL = 112

_mesh = plsc.VectorSubcoreMesh(
    core_axis_name="c", subcore_axis_name="s", num_cores=NC, num_subcores=NS
)

_cp = pltpu.CompilerParams()
if "needs_layout_passes" in pltpu.CompilerParams.__dataclass_fields__:
    _cp = dataclasses.replace(_cp, needs_layout_passes=False)


def _body(
    elems_hbm,
    energy_hbm,
    scales_hbm,
    shifts_hbm,
    lookup_hbm,
    out_hbm,
    elems_v,
    energy_v,
    out_v,
    lookup_v,
    scale_t,
    shift_t,
    cs_v,
    cb_v,
    te_i,
    te_f,
    to_f,
    sem_tbl,
    sem_a,
    sem_b,
    sem_tail,
    sem_out,
):
    wid = lax.axis_index("c") * NS + lax.axis_index("s")
    base = pl.multiple_of(wid * CHUNK, 16)
    base_b = pl.multiple_of(base + HALF_A, 16)
    is_tail = wid < TAIL_VECS
    tb = pl.multiple_of(TAIL_BASE + wid * L, 16)

    lookup_v[pl.ds(TBL - L, L)] = jnp.zeros((L,), jnp.int32)

    pltpu.async_copy(lookup_hbm, lookup_v.at[pl.ds(0, 100)], sem_tbl)
    pltpu.async_copy(scales_hbm, scale_t.at[pl.ds(0, 99)], sem_tbl)
    pltpu.async_copy(shifts_hbm, shift_t.at[pl.ds(0, 99)], sem_tbl)
    pltpu.async_copy(elems_hbm.at[pl.ds(base, HALF_A)], elems_v.at[pl.ds(0, HALF_A)], sem_a)
    pltpu.async_copy(energy_hbm.at[pl.ds(base, HALF_A)], energy_v.at[pl.ds(0, HALF_A)], sem_a)
    pltpu.async_copy(elems_hbm.at[pl.ds(base_b, HALF_B)], elems_v.at[pl.ds(HALF_A, HALF_B)], sem_b)
    pltpu.async_copy(energy_hbm.at[pl.ds(base_b, HALF_B)], energy_v.at[pl.ds(HALF_A, HALF_B)], sem_b)

    @pl.when(is_tail)
    def _():
        pltpu.async_copy(elems_hbm.at[pl.ds(tb, L)], te_i, sem_tail)
        pltpu.async_copy(energy_hbm.at[pl.ds(tb, L)], te_f, sem_tail)

    pltpu.make_async_copy(lookup_hbm, lookup_v.at[pl.ds(0, 100)], sem_tbl).wait()
    pltpu.make_async_copy(scales_hbm, scale_t.at[pl.ds(0, 99)], sem_tbl).wait()
    pltpu.make_async_copy(shifts_hbm, shift_t.at[pl.ds(0, 99)], sem_tbl).wait()
    for e0 in range(0, TBL, L):
        lv = lookup_v[pl.ds(e0, L)]
        cs_v[pl.ds(e0, L)] = plsc.load_gather(scale_t, [lv])
        cb_v[pl.ds(e0, L)] = plsc.load_gather(shift_t, [lv])

    def fma_vec(i):
        o = i * L
        ev = elems_v[pl.ds(o, L)]
        en = energy_v[pl.ds(o, L)]
        out_v[pl.ds(o, L)] = (
            en * plsc.load_gather(cs_v, [ev]) + plsc.load_gather(cb_v, [ev])
        )

    pltpu.make_async_copy(elems_hbm.at[pl.ds(base, HALF_A)], elems_v.at[pl.ds(0, HALF_A)], sem_a).wait()
    pltpu.make_async_copy(energy_hbm.at[pl.ds(base, HALF_A)], energy_v.at[pl.ds(0, HALF_A)], sem_a).wait()
    pl.loop(0, HALF_VECS_A, unroll=7)(fma_vec)
    pltpu.async_copy(out_v.at[pl.ds(0, HALF_A)], out_hbm.at[pl.ds(base, HALF_A)], sem_out)

    pltpu.make_async_copy(elems_hbm.at[pl.ds(base_b, HALF_B)], elems_v.at[pl.ds(HALF_A, HALF_B)], sem_b).wait()
    pltpu.make_async_copy(energy_hbm.at[pl.ds(base_b, HALF_B)], energy_v.at[pl.ds(HALF_A, HALF_B)], sem_b).wait()
    pl.loop(HALF_VECS_A, MAIN_VECS, unroll=7)(fma_vec)
    pltpu.async_copy(out_v.at[pl.ds(HALF_A, HALF_B)], out_hbm.at[pl.ds(base_b, HALF_B)], sem_out)

    @pl.when(is_tail)
    def _():
        pltpu.make_async_copy(elems_hbm.at[pl.ds(tb, L)], te_i, sem_tail).wait()
        pltpu.make_async_copy(energy_hbm.at[pl.ds(tb, L)], te_f, sem_tail).wait()
        ev = te_i[...]
        to_f[...] = (
            te_f[...] * plsc.load_gather(cs_v, [ev])
            + plsc.load_gather(cb_v, [ev])
        )
        pltpu.sync_copy(to_f, out_hbm.at[pl.ds(tb, L)])

    pltpu.make_async_copy(out_v.at[pl.ds(0, HALF_A)], out_hbm.at[pl.ds(base, HALF_A)], sem_out).wait()
    pltpu.make_async_copy(out_v.at[pl.ds(HALF_A, HALF_B)], out_hbm.at[pl.ds(base_b, HALF_B)], sem_out).wait()


def kernel(elems, atomic_energy, scales, shifts, elem_lookup):
    k = pl.kernel(
        _body,
        out_type=jax.ShapeDtypeStruct((N,), jnp.float32),
        mesh=_mesh,
        compiler_params=_cp,
        scratch_types=[
            pltpu.VMEM((CHUNK,), jnp.int32),
            pltpu.VMEM((CHUNK,), jnp.float32),
            pltpu.VMEM((CHUNK,), jnp.float32),
            pltpu.VMEM((TBL,), jnp.int32),
            pltpu.VMEM((TBL,), jnp.float32),
            pltpu.VMEM((TBL,), jnp.float32),
            pltpu.VMEM((TBL,), jnp.float32),
            pltpu.VMEM((TBL,), jnp.float32),
            pltpu.VMEM((L,), jnp.int32),
            pltpu.VMEM((L,), jnp.float32),
            pltpu.VMEM((L,), jnp.float32),
            pltpu.SemaphoreType.DMA,
            pltpu.SemaphoreType.DMA,
            pltpu.SemaphoreType.DMA,
            pltpu.SemaphoreType.DMA,
            pltpu.SemaphoreType.DMA,
        ],
    )
    return k(elems, atomic_energy, scales, shifts, elem_lookup)


# speedup vs baseline: 1.0021x; 1.0021x over previous
import dataclasses

import jax
import jax.numpy as jnp
from jax import lax
from jax.experimental import pallas as pl
from jax.experimental.pallas import tpu as pltpu
from jax.experimental.pallas import tpu_sc as plsc

N = 100000
NC, NS, L = 2, 16, 16
NW = NC * NS
VECS = N // L
STRIDE_VECS = VECS // NW
STRIDE = STRIDE_VECS * L
WORK_VECS = VECS - (NW - 1) * STRIDE_VECS
WORK = WORK_VECS * L
TBL = 112

_mesh = plsc.VectorSubcoreMesh(
    core_axis_name="c", subcore_axis_name="s", num_cores=NC, num_subcores=NS
)

_cp = pltpu.CompilerParams()
if "needs_layout_passes" in pltpu.CompilerParams.__dataclass_fields__:
    _cp = dataclasses.replace(_cp, needs_layout_passes=False)


def _body(
    elems_hbm,
    energy_hbm,
    scales_hbm,
    shifts_hbm,
    lookup_hbm,
    out_hbm,
    elems_v,
    energy_v,
    out_v,
    lookup_v,
    scale_t,
    shift_t,
    cs_v,
    cb_v,
    sem_tbl,
    sem_in,
    sem_out,
):
    wid = lax.axis_index("c") * NS + lax.axis_index("s")
    base = pl.multiple_of(wid * STRIDE, 16)

    lookup_v[pl.ds(TBL - L, L)] = jnp.zeros((L,), jnp.int32)

    pltpu.async_copy(lookup_hbm, lookup_v.at[pl.ds(0, 100)], sem_tbl)
    pltpu.async_copy(scales_hbm, scale_t.at[pl.ds(0, 99)], sem_tbl)
    pltpu.async_copy(shifts_hbm, shift_t.at[pl.ds(0, 99)], sem_tbl)
    pltpu.async_copy(elems_hbm.at[pl.ds(base, WORK)], elems_v, sem_in)
    pltpu.async_copy(energy_hbm.at[pl.ds(base, WORK)], energy_v, sem_in)

    pltpu.make_async_copy(lookup_hbm, lookup_v.at[pl.ds(0, 100)], sem_tbl).wait()
    pltpu.make_async_copy(scales_hbm, scale_t.at[pl.ds(0, 99)], sem_tbl).wait()
    pltpu.make_async_copy(shifts_hbm, shift_t.at[pl.ds(0, 99)], sem_tbl).wait()
    for e0 in range(0, TBL, L):
        lv = lookup_v[pl.ds(e0, L)]
        cs_v[pl.ds(e0, L)] = plsc.load_gather(scale_t, [lv])
        cb_v[pl.ds(e0, L)] = plsc.load_gather(shift_t, [lv])

    pltpu.make_async_copy(elems_hbm.at[pl.ds(base, WORK)], elems_v, sem_in).wait()
    pltpu.make_async_copy(energy_hbm.at[pl.ds(base, WORK)], energy_v, sem_in).wait()

    @pl.loop(0, WORK_VECS, unroll=5)
    def _(i):
        o = i * L
        ev = elems_v[pl.ds(o, L)]
        en = energy_v[pl.ds(o, L)]
        out_v[pl.ds(o, L)] = (
            en * plsc.load_gather(cs_v, [ev]) + plsc.load_gather(cb_v, [ev])
        )

    pltpu.async_copy(out_v, out_hbm.at[pl.ds(base, WORK)], sem_out)
    pltpu.make_async_copy(out_v, out_hbm.at[pl.ds(base, WORK)], sem_out).wait()


def kernel(elems, atomic_energy, scales, shifts, elem_lookup):
    k = pl.kernel(
        _body,
        out_type=jax.ShapeDtypeStruct((N,), jnp.float32),
        mesh=_mesh,
        compiler_params=_cp,
        scratch_types=[
            pltpu.VMEM((WORK,), jnp.int32),
            pltpu.VMEM((WORK,), jnp.float32),
            pltpu.VMEM((WORK,), jnp.float32),
            pltpu.VMEM((TBL,), jnp.int32),
            pltpu.VMEM((TBL,), jnp.float32),
            pltpu.VMEM((TBL,), jnp.float32),
            pltpu.VMEM((TBL,), jnp.float32),
            pltpu.VMEM((TBL,), jnp.float32),
            pltpu.SemaphoreType.DMA,
            pltpu.SemaphoreType.DMA,
            pltpu.SemaphoreType.DMA,
        ],
    )
    return k(elems, atomic_energy, scales, shifts, elem_lookup)
